# baseline (speedup 1.0000x reference)
import functools

import jax
import jax.numpy as jnp
from jax import lax
from jax.experimental import pallas as pl
from jax.experimental.pallas import tpu as pltpu
from jax.experimental.pallas import tpu_sc as plsc

L = 16
NC = 2
NS = 16
NW = NC * NS
CHUNK = 128


def _mf_body(usr_idx_hbm, itm_idx_hbm, usr_nid_hbm, itm_nid_hbm,
             usr_table_hbm, itm_table_hbm, out_hbm,
             uidx_v, iidx_v, cu_v, ci_v, urows_v, irows_v, out_v,
             sem_idx, sem_rows):
    wid = lax.axis_index("s") * NC + lax.axis_index("c")
    nchunk = uidx_v.shape[0]
    epw = nchunk * CHUNK
    base_row = wid * nchunk

    pltpu.sync_copy(usr_idx_hbm.at[pl.ds(base_row, nchunk)], uidx_v)
    pltpu.sync_copy(itm_idx_hbm.at[pl.ds(base_row, nchunk)], iidx_v)

    cps = []
    for c in range(nchunk):
        cps.append(pltpu.async_copy(
            usr_nid_hbm.at[uidx_v.at[c]], cu_v.at[c], sem_idx))
        cps.append(pltpu.async_copy(
            itm_nid_hbm.at[iidx_v.at[c]], ci_v.at[c], sem_idx))
    for cp in cps:
        cp.wait()

    cps = []
    for c in range(nchunk):
        cps.append(pltpu.async_copy(
            usr_table_hbm.at[cu_v.at[c]],
            urows_v.at[pl.ds(c * CHUNK, CHUNK)], sem_rows))
        cps.append(pltpu.async_copy(
            itm_table_hbm.at[ci_v.at[c]],
            irows_v.at[pl.ds(c * CHUNK, CHUNK)], sem_rows))
    for cp in cps:
        cp.wait()

    lanes = lax.iota(jnp.int32, L)

    perm_dnums = lax.GatherDimensionNumbers(
        offset_dims=(), collapsed_slice_dims=(0,), start_index_map=(0,))

    def vperm(v, idx):
        return lax.gather(v, idx[:, None], perm_dnums, (1,),
                          mode=lax.GatherScatterMode.PROMISE_IN_BOUNDS)

    perms = [(lanes + sh) % L for sh in (8, 4, 2, 1)]

    def group(g, carry):
        base = g * L
        sv = jnp.zeros((L,), jnp.float32)
        for j in range(L):
            r = base + j
            p = urows_v[r, :] * irows_v[r, :]
            for perm in perms:
                p = p + vperm(p, perm)
            sv = jnp.where(lanes == j, p, sv)
        out_v[pl.ds(base, L)] = sv
        return carry

    lax.fori_loop(0, epw // L, group, 0)

    pltpu.sync_copy(out_v, out_hbm.at[pl.ds(wid * epw, epw)])


def kernel(usr_n_id, itm_n_id, edge_label_index, usr_table, itm_table):
    B = usr_n_id.shape[0]
    epw = B // NW
    nchunk = epw // CHUNK

    usr_idx = edge_label_index[0].astype(jnp.int32).reshape(B // CHUNK, CHUNK)
    itm_idx = edge_label_index[1].astype(jnp.int32).reshape(B // CHUNK, CHUNK)
    usr_n_id = usr_n_id.astype(jnp.int32)
    itm_n_id = itm_n_id.astype(jnp.int32)

    mesh = plsc.VectorSubcoreMesh(core_axis_name="c", subcore_axis_name="s")
    f = functools.partial(
        pl.kernel,
        mesh=mesh,
        compiler_params=pltpu.CompilerParams(use_tc_tiling_on_sc=False),
        out_type=jax.ShapeDtypeStruct((B,), jnp.float32),
        scratch_types=[
            pltpu.VMEM((nchunk, CHUNK), jnp.int32),
            pltpu.VMEM((nchunk, CHUNK), jnp.int32),
            pltpu.VMEM((nchunk, CHUNK), jnp.int32),
            pltpu.VMEM((nchunk, CHUNK), jnp.int32),
            pltpu.VMEM((epw, L), jnp.float32),
            pltpu.VMEM((epw, L), jnp.float32),
            pltpu.VMEM((epw,), jnp.float32),
            pltpu.SemaphoreType.DMA,
            pltpu.SemaphoreType.DMA,
        ],
    )(_mf_body)
    return f(usr_idx, itm_idx, usr_n_id, itm_n_id, usr_table, itm_table)
